# baseline (speedup 1.0000x reference)
import jax
import jax.numpy as jnp
from jax.experimental import pallas as pl
from jax.experimental.pallas import tpu as pltpu
from jax.experimental.pallas import tpu_sc as plsc

_B, _S, _D = 8, 576, 2048
_K, _DQ = 100000, 512
_KB = 4000
_NKB = _K // _KB


def _argmin_body(keys_ref, qt_ref, out_ref, minv_ref, mini_ref):
    i = pl.program_id(0)
    kb = keys_ref[...]
    cross = jnp.dot(kb, qt_ref[...],
                    preferred_element_type=jnp.float32,
                    precision=jax.lax.Precision.HIGHEST)
    k2 = jnp.sum(kb * kb, axis=1, keepdims=True)
    d2 = k2 - 2.0 * cross
    bmin = jnp.min(d2, axis=0, keepdims=True)
    rows = jax.lax.broadcasted_iota(jnp.int32, d2.shape, 0)
    barg = jnp.min(jnp.where(d2 == bmin, rows, _K),
                   axis=0, keepdims=True) + i * _KB

    @pl.when(i == 0)
    def _():
        minv_ref[...] = bmin
        mini_ref[...] = barg

    @pl.when(i > 0)
    def _():
        better = bmin < minv_ref[...]
        minv_ref[...] = jnp.where(better, bmin, minv_ref[...])
        mini_ref[...] = jnp.where(better, barg, mini_ref[...])

    @pl.when(i == _NKB - 1)
    def _():
        out_ref[...] = mini_ref[...]


def _mm_body(h_ref, w_ref, b_ref, v_ref, out_ref):
    acc = jnp.dot(h_ref[0], w_ref[...],
                  preferred_element_type=jnp.float32)
    out_ref[0, :_S, :] = acc + b_ref[...]
    v = v_ref[0]
    s = jnp.sum(v * v, axis=1, keepdims=True)
    nrm = jnp.maximum(jnp.sqrt(s), 1e-12)
    out_ref[0, _S:, :] = v / nrm


def _sc_gather(values, idx):
    mesh = plsc.VectorSubcoreMesh(core_axis_name="core",
                                  subcore_axis_name="subcore")

    idx16 = jnp.concatenate([idx, jnp.zeros((1, 16 - _B), jnp.int32)], axis=1)

    @pl.kernel(out_type=jax.ShapeDtypeStruct((16, _D), jnp.float32),
               mesh=mesh)
    def g(values_hbm, i_hbm, o_hbm):
        def body(i_vmem, o_vmem):
            pltpu.sync_copy(values_hbm.at[i_vmem.at[0]], o_vmem)

        pltpu.emit_pipeline(
            body,
            grid=(1,),
            in_specs=[pl.BlockSpec((1, 16), lambda i: (0, 0))],
            out_specs=[pl.BlockSpec((16, _D), lambda i: (0, 0))],
            core_axis_name=("core", "subcore"),
            dimension_semantics=(pltpu.PARALLEL,),
        )(i_hbm, o_hbm)

    return g(values, idx16)[:_B]


def kernel(hidden_state, concept_signal, W, b, keys_mat, values):
    qt = concept_signal[:, 0, :].T

    idx = pl.pallas_call(
        _argmin_body,
        grid=(_NKB,),
        in_specs=[pl.BlockSpec((_KB, _DQ), lambda i: (i, 0)),
                  pl.BlockSpec((_DQ, _B), lambda i: (0, 0))],
        out_specs=pl.BlockSpec((1, _B), lambda i: (0, 0)),
        out_shape=jax.ShapeDtypeStruct((1, _B), jnp.int32),
        scratch_shapes=[pltpu.VMEM((1, _B), jnp.float32),
                        pltpu.VMEM((1, _B), jnp.int32)],
    )(keys_mat, qt)

    vrows = _sc_gather(values, idx).reshape(_B, 1, _D)

    out = pl.pallas_call(
        _mm_body,
        grid=(_B,),
        in_specs=[pl.BlockSpec((1, _S, _D), lambda i: (i, 0, 0)),
                  pl.BlockSpec((_D, _D), lambda i: (0, 0)),
                  pl.BlockSpec((1, _D), lambda i: (0, 0)),
                  pl.BlockSpec((1, 1, _D), lambda i: (i, 0, 0))],
        out_specs=pl.BlockSpec((1, _S + 1, _D), lambda i: (i, 0, 0)),
        out_shape=jax.ShapeDtypeStruct((_B, _S + 1, _D), jnp.float32),
    )(hidden_state, W, b.reshape(1, _D), vrows)
    return out
